# Initial kernel scaffold; baseline (speedup 1.0000x reference)
import jax
import jax.numpy as jnp
from jax.experimental import pallas as pl


def kernel(T, bias_table):
    raise NotImplementedError("write your pallas kernel here")


# speedup vs baseline: 41.8934x; 41.8934x over previous
import functools

import jax
import jax.numpy as jnp
from jax import lax
from jax.experimental import pallas as pl
from jax.experimental.pallas import tpu as pltpu
from jax.experimental.pallas import tpu_sc as plsc

NUM_HEADS = 16
T_STATIC = 2048
NUM_BUCKETS = 257
LANES = 16
NUM_REPLICAS = 8
REPLICA_PITCH = 4096
GROUP = 8


def _sc_body(table_hbm, out_hbm, table_v, w1_v, sem):
    c = lax.axis_index("c")
    s = lax.axis_index("s")
    h = s

    pltpu.sync_copy(table_hbm, table_v)

    hvec = jnp.full((LANES,), h, dtype=jnp.int32)
    iot = lax.iota(jnp.int32, LANES)

    def build(ci, _):
        base = ci * LANES
        pos = base + iot
        for p in range(NUM_REPLICAS):
            idxr = jnp.clip(2168 - pos + p, 0, NUM_BUCKETS - 1)
            val = plsc.load_gather(table_v, [idxr, hvec])
            w1_v[pl.ds(pl.multiple_of(p * REPLICA_PITCH + base, LANES), LANES)] = val
        return _

    lax.fori_loop(0, REPLICA_PITCH // LANES, build, None)

    row_base = (h * T_STATIC + c * 1024) * T_STATIC

    def emit(g, _):
        i0 = g * GROUP + c * 1024
        descs = []
        for u in range(GROUP):
            src_start = pl.multiple_of(u * REPLICA_PITCH + 2040 - i0, 8)
            src = w1_v.at[pl.ds(src_start, T_STATIC)]
            dst = out_hbm.at[pl.ds(row_base + (g * GROUP + u) * T_STATIC, T_STATIC)]
            descs.append(pltpu.async_copy(src, dst, sem))
        for d in descs:
            d.wait()
        return _

    lax.fori_loop(0, 1024 // GROUP, emit, None)


@jax.jit
def _run(bias_table):
    mesh = plsc.VectorSubcoreMesh(
        core_axis_name="c", subcore_axis_name="s", num_cores=2, num_subcores=16
    )
    f = pl.kernel(
        _sc_body,
        out_type=jax.ShapeDtypeStruct(
            (NUM_HEADS * T_STATIC * T_STATIC,), jnp.float32
        ),
        mesh=mesh,
        scratch_types=[
            pltpu.VMEM((NUM_BUCKETS, NUM_HEADS), jnp.float32),
            pltpu.VMEM((NUM_REPLICAS * REPLICA_PITCH,), jnp.float32),
            pltpu.SemaphoreType.DMA,
        ],
        compiler_params=pltpu.CompilerParams(needs_layout_passes=False),
    )
    flat = f(bias_table)
    return flat.reshape(1, NUM_HEADS, T_STATIC, T_STATIC)


def kernel(T, bias_table):
    return _run(bias_table)


# speedup vs baseline: 110.6129x; 2.6403x over previous
import jax
import jax.numpy as jnp
from jax import lax
from jax.experimental import pallas as pl
from jax.experimental.pallas import tpu as pltpu
from jax.experimental.pallas import tpu_sc as plsc

NUM_HEADS = 16
T_STATIC = 2048
NUM_BUCKETS = 257
LANES = 16
NUM_COLS = 23
R_MINOR = NUM_COLS * 128


def _sc_body(table_hbm, out_hbm, table_v, r_v, sem):
    c = lax.axis_index("c")
    s = lax.axis_index("s")
    h = s
    a0 = c * 128

    pltpu.sync_copy(table_hbm, table_v)

    hvec = jnp.full((LANES,), h, dtype=jnp.int32)
    iot = lax.iota(jnp.int32, LANES)

    def fill_column(jj, t):
        col0 = jj * 128
        for rr in range(8):
            for u in range(8):
                ll = u * LANES + iot
                bucket = jnp.clip(8 * t + rr - ll, -128, 128) + 128
                val = plsc.load_gather(table_v, [bucket, hvec])
                off = pl.multiple_of(col0 + u * LANES, LANES)
                r_v[rr, pl.ds(off, LANES)] = val

    def init(jj, _):
        fill_column(jj, a0 + 112 - 16 * jj)
        return _

    lax.fori_loop(0, NUM_COLS, init, None)

    def emit(r, _):
        for dcol in range(4):
            jj = c * 8 + 5 + dcol
            t = r + 32 - 16 * dcol
            fill_column(jj, t)
        descs = []
        for k in range(8):
            a = a0 + r + 16 * k
            src = r_v.at[:, pl.ds(128 * (7 - k), T_STATIC)]
            dst = out_hbm.at[0, h, pl.ds(pl.multiple_of(8 * a, 8), 8), :]
            descs.append(pltpu.async_copy(src, dst, sem))
        for d in descs:
            d.wait()
        return _

    lax.fori_loop(0, 16, emit, None)


@jax.jit
def _run(bias_table):
    mesh = plsc.VectorSubcoreMesh(
        core_axis_name="c", subcore_axis_name="s", num_cores=2, num_subcores=16
    )
    f = pl.kernel(
        _sc_body,
        out_type=jax.ShapeDtypeStruct(
            (1, NUM_HEADS, T_STATIC, T_STATIC), jnp.float32
        ),
        mesh=mesh,
        scratch_types=[
            pltpu.VMEM((NUM_BUCKETS, NUM_HEADS), jnp.float32),
            pltpu.VMEM((8, R_MINOR), jnp.float32),
            pltpu.SemaphoreType.DMA,
        ],
        compiler_params=pltpu.CompilerParams(needs_layout_passes=False),
    )
    return f(bias_table)


def kernel(T, bias_table):
    return _run(bias_table)
